# baseline (speedup 1.0000x reference)
import jax
import jax.numpy as jnp
from jax import lax
from jax.experimental import pallas as pl
from jax.experimental.pallas import tpu as pltpu
from jax.experimental.pallas import tpu_sc as plsc

B, L, D = 1024, 200, 768
VL = 16
NJ = D // VL
NC, NS = 2, 16
NW = NC * NS
RPW = B // NW
C = 40
NCH = L // C
EPS = 1e-12


def _encoder_body(ids_hbm, tt_hbm, we_hbm, pe_hbm, te_hbm, lnw_hbm, lnb_hbm,
                  out_hbm, ids_v, tt_v, posc_v, rows_v, par_v, sem):
    wid = lax.axis_index("s") * NC + lax.axis_index("c")
    b0 = wid * RPW

    pltpu.sync_copy(ids_hbm.at[pl.ds(b0 * L, RPW * L)], ids_v)
    pltpu.sync_copy(tt_hbm.at[pl.ds(b0 * L, RPW * L)],
                    tt_v.at[pl.ds(0, RPW * L)])

    pltpu.sync_copy(te_hbm.at[pl.ds(0, 8)], par_v)
    for j in range(NJ):
        sl = pl.ds(j * VL, VL)
        par_v[1, sl] = par_v[1, sl] - par_v[0, sl]
    pltpu.sync_copy(lnw_hbm, par_v.at[2])
    pltpu.sync_copy(lnb_hbm, par_v.at[3])

    for lc in range(NCH):
        l0 = lc * C
        pltpu.sync_copy(pe_hbm.at[pl.ds(l0, C)], posc_v)

        def fold_body(t, _):
            for j in range(NJ):
                sl = pl.ds(j * VL, VL)
                posc_v[t, sl] = posc_v[t, sl] + par_v[0, sl]
            return 0
        lax.fori_loop(0, C, fold_body, 0)

        def row_body(i, _):
            pltpu.async_copy(
                we_hbm.at[ids_v.at[pl.ds(i * L + l0, C)]], rows_v, sem
            ).wait()

            def tok_body(t, _):
                ttf = tt_v[pl.ds(i * L + l0 + t, VL)][0].astype(jnp.float32)
                acc = jnp.zeros((VL,), jnp.float32)
                acc2 = jnp.zeros((VL,), jnp.float32)
                for j in range(NJ):
                    sl = pl.ds(j * VL, VL)
                    v = rows_v[t, sl] + (posc_v[t, sl] + ttf * par_v[1, sl])
                    rows_v[t, sl] = v
                    acc = acc + v
                    acc2 = acc2 + v * v
                def lanesum(v):
                    dnums = lax.GatherDimensionNumbers(
                        offset_dims=(), collapsed_slice_dims=(0,),
                        start_index_map=(0,))
                    for sh in (8, 4, 2, 1):
                        idx = lax.iota(jnp.int32, VL) ^ sh
                        v = v + lax.gather(
                            v, idx[:, None], dnums, (1,),
                            mode=lax.GatherScatterMode.PROMISE_IN_BOUNDS)
                    return v

                meanv = lanesum(acc) * (1.0 / D)
                x = lanesum(acc2) * (1.0 / D) - meanv * meanv + EPS
                yi = 0x5F3759DF - lax.shift_right_logical(
                    lax.bitcast_convert_type(x, jnp.int32), 1)
                y = lax.bitcast_convert_type(yi, jnp.float32)
                for _ in range(3):
                    y = y * (1.5 - 0.5 * x * y * y)
                for j in range(NJ):
                    sl = pl.ds(j * VL, VL)
                    rows_v[t, sl] = ((rows_v[t, sl] - meanv) * y
                                     * par_v[2, sl] + par_v[3, sl])
                return 0
            lax.fori_loop(0, C, tok_body, 0)

            pltpu.sync_copy(rows_v, out_hbm.at[b0 + i, pl.ds(l0, C)])
            return 0
        lax.fori_loop(0, RPW, row_body, 0)


def kernel(input_ids, token_type_ids, word_embeddings, position_embeddings,
           token_type_embeddings, ln_weight, ln_bias):
    enc = pl.kernel(
        _encoder_body,
        out_type=jax.ShapeDtypeStruct((B, L, D), jnp.float32),
        mesh=plsc.VectorSubcoreMesh(core_axis_name="c", subcore_axis_name="s",
                                    num_cores=NC, num_subcores=NS),
        scratch_types=[
            pltpu.VMEM((RPW * L,), jnp.int32),
            pltpu.VMEM((RPW * L + VL,), jnp.int32),
            pltpu.VMEM((C, D), jnp.float32),
            pltpu.VMEM((C, D), jnp.float32),
            pltpu.VMEM((8, D), jnp.float32),
            pltpu.SemaphoreType.DMA,
        ],
    )
    return enc(input_ids.reshape(B * L), token_type_ids.reshape(B * L),
               word_embeddings, position_embeddings, token_type_embeddings,
               ln_weight, ln_bias)
